# baseline (speedup 1.0000x reference)
import jax
import jax.numpy as jnp
from jax.experimental import pallas as pl
from jax.experimental.pallas import tpu as pltpu

_K = 8


def kernel(x, pe):
    b, s, d_x = x.shape
    size, dim = pe.shape
    d_o = d_x + dim
    rows = b * s
    chunk = rows // _K
    x2 = x.reshape(rows, d_x)

    def body(x_hbm, pe_hbm, o_hbm, pe_buf, sem_pe, sems_x, sems_pe_out):
        cp_pe = pltpu.async_copy(pe_hbm, pe_buf, sem_pe)
        x_copies = []
        for i in range(_K):
            x_copies.append(
                pltpu.async_copy(
                    x_hbm.at[pl.ds(i * chunk, chunk), :],
                    o_hbm.at[pl.ds(i * chunk, chunk), pl.ds(0, d_x)],
                    sems_x.at[i],
                )
            )
        cp_pe.wait()
        pe_copies = []
        for j in range(b):
            pe_copies.append(
                pltpu.async_copy(
                    pe_buf,
                    o_hbm.at[pl.ds(j * size, size), pl.ds(d_x, dim)],
                    sems_pe_out.at[j],
                )
            )
        for c in x_copies:
            c.wait()
        for c in pe_copies:
            c.wait()

    out2 = pl.pallas_call(
        body,
        in_specs=[
            pl.BlockSpec(memory_space=pltpu.MemorySpace.HBM),
            pl.BlockSpec(memory_space=pltpu.MemorySpace.HBM),
        ],
        out_specs=pl.BlockSpec(memory_space=pltpu.MemorySpace.HBM),
        out_shape=jax.ShapeDtypeStruct((rows, d_o), x.dtype),
        scratch_shapes=[
            pltpu.VMEM((size, dim), x.dtype),
            pltpu.SemaphoreType.DMA,
            pltpu.SemaphoreType.DMA((_K,)),
            pltpu.SemaphoreType.DMA((b,)),
        ],
    )(x2, pe)
    return out2.reshape(b, s, d_o)


# speedup vs baseline: 45.5524x; 45.5524x over previous
import jax
import jax.numpy as jnp
from jax.experimental import pallas as pl

_D_X = 1024
_S_BLK = 2048


def _concat_body(x_ref, pe_ref, o_ref):
    j = pl.program_id(1)
    o_ref[:, :, :_D_X] = x_ref[...]
    o_ref[:, :, _D_X:] = pe_ref[pl.ds(j * _S_BLK, _S_BLK), :][None, :, :]


def kernel(x, pe):
    b, s, d_x = x.shape
    size, dim = pe.shape
    grid = (b, s // _S_BLK)
    return pl.pallas_call(
        _concat_body,
        grid=grid,
        in_specs=[
            pl.BlockSpec((1, _S_BLK, d_x), lambda i, j: (i, j, 0)),
            pl.BlockSpec((size, dim), lambda i, j: (0, 0)),
        ],
        out_specs=pl.BlockSpec((1, _S_BLK, d_x + dim), lambda i, j: (i, j, 0)),
        out_shape=jax.ShapeDtypeStruct((b, s, d_x + dim), x.dtype),
    )(x, pe)
